# initial kernel scaffold (unmeasured)
import jax
import jax.numpy as jnp
from jax import lax
from jax.experimental import pallas as pl
from jax.experimental.pallas import tpu as pltpu

N_DEV = 4
SQ = 256
SKV = 4096
H_LOC = 8
DH = 128
DM = 1024
BLK = 64
SCALE = 0.08838834764831843


def kernel(x, Wq, K_ext, V_ext, Wo):
    def body(x_ref, wq_ref, k_hbm, v_hbm, wo_ref, out_ref,
             k_buf, v_buf, k_sem, v_sem, comm_ref, send_sems, recv_sems):
        my = lax.axis_index("i")
        p1 = my ^ 1
        p2 = 3 - my

        barrier_sem = pltpu.get_barrier_semaphore()
        for nbr in (p1, p2):
            pl.semaphore_signal(barrier_sem, inc=1, device_id=(nbr,),
                                device_id_type=pl.DeviceIdType.MESH)
        pl.semaphore_wait(barrier_sem, 2)

        qb = lax.broadcasted_iota(jnp.int32, (SQ, SKV), 0) // BLK
        kb = lax.broadcasted_iota(jnp.int32, (SQ, SKV), 1) // BLK
        mask = (qb == kb) | (kb == 0) | ((qb + kb) % 3 == 0)

        x2 = x_ref[0]
        acc = jnp.zeros((SQ, DM), jnp.float32)
        for h in range(H_LOC):
            head = my * H_LOC + h
            ck = pltpu.make_async_copy(
                k_hbm.at[0, :, pl.ds(head, 1), :], k_buf, k_sem)
            cv = pltpu.make_async_copy(
                v_hbm.at[0, :, pl.ds(head, 1), :], v_buf, v_sem)
            ck.start()
            cv.start()
            ck.wait()
            cv.wait()
            k = k_buf[:, 0, :]
            v = v_buf[:, 0, :]

            qh = jnp.dot(x2, wq_ref[:, h * DH:(h + 1) * DH],
                         preferred_element_type=jnp.float32)
            s = lax.dot_general(qh, k, (((1,), (1,)), ((), ())),
                                preferred_element_type=jnp.float32) * SCALE
            s = jnp.where(mask, s, -1e9)
            m = jnp.max(s, axis=1, keepdims=True)
            e = jnp.exp(s - m)
            w = e / jnp.sum(e, axis=1, keepdims=True)
            ctx = jnp.dot(w, v, preferred_element_type=jnp.float32)
            acc = acc + jnp.dot(ctx, wo_ref[h * DH:(h + 1) * DH, :],
                                preferred_element_type=jnp.float32)

        comm_ref[0] = acc
        rdma1 = pltpu.make_async_remote_copy(
            src_ref=comm_ref.at[0], dst_ref=comm_ref.at[1],
            send_sem=send_sems.at[0], recv_sem=recv_sems.at[0],
            device_id=(p1,), device_id_type=pl.DeviceIdType.MESH)
        rdma1.start()
        rdma1.wait()
        acc = acc + comm_ref[1]

        comm_ref[0] = acc
        rdma2 = pltpu.make_async_remote_copy(
            src_ref=comm_ref.at[0], dst_ref=comm_ref.at[2],
            send_sem=send_sems.at[1], recv_sem=recv_sems.at[1],
            device_id=(p2,), device_id_type=pl.DeviceIdType.MESH)
        rdma2.start()
        rdma2.wait()
        out_ref[0] = acc + comm_ref[2]

    return pl.pallas_call(
        body,
        out_shape=jax.ShapeDtypeStruct((1, SQ, DM), jnp.float32),
        in_specs=[
            pl.BlockSpec(memory_space=pltpu.VMEM),
            pl.BlockSpec(memory_space=pltpu.VMEM),
            pl.BlockSpec(memory_space=pltpu.ANY),
            pl.BlockSpec(memory_space=pltpu.ANY),
            pl.BlockSpec(memory_space=pltpu.VMEM),
        ],
        out_specs=pl.BlockSpec(memory_space=pltpu.VMEM),
        scratch_shapes=[
            pltpu.VMEM((SKV, 1, DH), jnp.float32),
            pltpu.VMEM((SKV, 1, DH), jnp.float32),
            pltpu.SemaphoreType.DMA,
            pltpu.SemaphoreType.DMA,
            pltpu.VMEM((3, SQ, DM), jnp.float32),
            pltpu.SemaphoreType.DMA((2,)),
            pltpu.SemaphoreType.DMA((2,)),
        ],
        compiler_params=pltpu.CompilerParams(collective_id=0),
    )(x, Wq, K_ext, V_ext, Wo)


# baseline (device time: 96626 ns/iter reference)
import jax
import jax.numpy as jnp
from jax import lax
from jax.experimental import pallas as pl
from jax.experimental.pallas import tpu as pltpu

N_DEV = 4
SQ = 256
SKV = 4096
H_LOC = 8
DH = 128
DM = 1024
BLK = 64
SCALE = 0.08838834764831843


def kernel(x, Wq, K_ext, V_ext, Wo):
    def body(x_ref, wq_ref, k_hbm, v_hbm, wo_ref, out_ref,
             k_buf, v_buf, k_sem, v_sem, comm_ref, send_sems, recv_sems):
        my = lax.axis_index("i")
        p1 = my ^ 1
        p2 = 3 - my

        barrier_sem = pltpu.get_barrier_semaphore()
        for nbr in (p1, p2):
            pl.semaphore_signal(barrier_sem, inc=1, device_id=(nbr,),
                                device_id_type=pl.DeviceIdType.MESH)
        pl.semaphore_wait(barrier_sem, 2)

        qb = lax.broadcasted_iota(jnp.int32, (SQ, SKV), 0) // BLK
        kb = lax.broadcasted_iota(jnp.int32, (SQ, SKV), 1) // BLK
        mask = (qb == kb) | (kb == 0) | ((qb + kb) % 3 == 0)

        x2 = x_ref[0]
        acc = jnp.zeros((SQ, DM), jnp.float32)
        for h in range(H_LOC):
            head = my * H_LOC + h
            ck = pltpu.make_async_copy(
                k_hbm.at[0, :, pl.ds(head, 1), :], k_buf, k_sem)
            cv = pltpu.make_async_copy(
                v_hbm.at[0, :, pl.ds(head, 1), :], v_buf, v_sem)
            ck.start()
            cv.start()
            ck.wait()
            cv.wait()
            k = k_buf[:, 0, :]
            v = v_buf[:, 0, :]

            qh = jnp.dot(x2, wq_ref[:, h * DH:(h + 1) * DH],
                         preferred_element_type=jnp.float32)
            s = lax.dot_general(qh, k, (((1,), (1,)), ((), ())),
                                preferred_element_type=jnp.float32) * SCALE
            s = jnp.where(mask, s, -1e9)
            m = jnp.max(s, axis=1, keepdims=True)
            e = jnp.exp(s - m)
            w = e / jnp.sum(e, axis=1, keepdims=True)
            ctx = jnp.dot(w, v, preferred_element_type=jnp.float32)
            acc = acc + jnp.dot(ctx, wo_ref[h * DH:(h + 1) * DH, :],
                                preferred_element_type=jnp.float32)

        comm_ref[0] = acc
        rdma1 = pltpu.make_async_remote_copy(
            src_ref=comm_ref.at[0], dst_ref=comm_ref.at[1],
            send_sem=send_sems.at[0], recv_sem=recv_sems.at[0],
            device_id=(p1,), device_id_type=pl.DeviceIdType.MESH)
        rdma1.start()
        rdma1.wait()
        acc = acc + comm_ref[1]

        comm_ref[0] = acc
        rdma2 = pltpu.make_async_remote_copy(
            src_ref=comm_ref.at[0], dst_ref=comm_ref.at[2],
            send_sem=send_sems.at[1], recv_sem=recv_sems.at[1],
            device_id=(p2,), device_id_type=pl.DeviceIdType.MESH)
        rdma2.start()
        rdma2.wait()
        out_ref[0] = acc + comm_ref[2]

    return pl.pallas_call(
        body,
        out_shape=jax.ShapeDtypeStruct((1, SQ, DM), jnp.float32),
        in_specs=[
            pl.BlockSpec(memory_space=pltpu.VMEM),
            pl.BlockSpec(memory_space=pltpu.VMEM),
            pl.BlockSpec(memory_space=pl.ANY),
            pl.BlockSpec(memory_space=pl.ANY),
            pl.BlockSpec(memory_space=pltpu.VMEM),
        ],
        out_specs=pl.BlockSpec(memory_space=pltpu.VMEM),
        scratch_shapes=[
            pltpu.VMEM((SKV, 1, DH), jnp.float32),
            pltpu.VMEM((SKV, 1, DH), jnp.float32),
            pltpu.SemaphoreType.DMA,
            pltpu.SemaphoreType.DMA,
            pltpu.VMEM((3, SQ, DM), jnp.float32),
            pltpu.SemaphoreType.DMA((2,)),
            pltpu.SemaphoreType.DMA((2,)),
        ],
        compiler_params=pltpu.CompilerParams(collective_id=0),
    )(x, Wq, K_ext, V_ext, Wo)


# device time: 50007 ns/iter; 1.9322x vs baseline; 1.9322x over previous
import jax
import jax.numpy as jnp
from jax import lax
from jax.experimental import pallas as pl
from jax.experimental.pallas import tpu as pltpu

N_DEV = 4
SQ = 256
SKV = 4096
H_LOC = 8
DH = 128
DM = 1024
BLK = 64
NQB = SQ // BLK
SCALE = 0.08838834764831843


def kernel(x, Wq, K_ext, V_ext, Wo):
    def body(x_ref, wq_ref, k_hbm, v_hbm, wo_ref, out_ref,
             k_buf, v_buf, k_sems, v_sems,
             s1buf, r1buf, s2buf, r2buf,
             s1send, s1recv, s2send, s2recv):
        my = lax.axis_index("i")
        p1 = my ^ 1
        p2 = 3 - my

        barrier_sem = pltpu.get_barrier_semaphore()
        for nbr in (p1, p2):
            pl.semaphore_signal(barrier_sem, inc=1, device_id=(nbr,),
                                device_id_type=pl.DeviceIdType.MESH)
        pl.semaphore_wait(barrier_sem, 2)

        k_copies, v_copies = [], []
        for h in range(H_LOC):
            head = my * H_LOC + h
            ck = pltpu.make_async_copy(
                k_hbm.at[0, :, head, :], k_buf.at[h], k_sems.at[h])
            cv = pltpu.make_async_copy(
                v_hbm.at[0, :, head, :], v_buf.at[h], v_sems.at[h])
            ck.start()
            cv.start()
            k_copies.append(ck)
            v_copies.append(cv)

        kb = lax.broadcasted_iota(jnp.int32, (1, SKV), 1) // BLK
        biases = []
        for qb in range(NQB):
            allow = (kb == qb) | (kb == 0) | ((qb + kb) % 3 == 0)
            biases.append(jnp.where(allow, 0.0, -1e9).astype(jnp.float32))

        q_all = jnp.dot(x_ref[0], wq_ref[...],
                        preferred_element_type=jnp.float32) * SCALE

        def rdma1(qb):
            return pltpu.make_async_remote_copy(
                src_ref=s1buf.at[qb], dst_ref=r1buf.at[qb],
                send_sem=s1send.at[qb], recv_sem=s1recv.at[qb],
                device_id=(p1,), device_id_type=pl.DeviceIdType.MESH)

        def rdma2(qb):
            return pltpu.make_async_remote_copy(
                src_ref=s2buf.at[qb], dst_ref=r2buf.at[qb],
                send_sem=s2send.at[qb], recv_sem=s2recv.at[qb],
                device_id=(p2,), device_id_type=pl.DeviceIdType.MESH)

        def stage2_issue(qb):
            rdma1(qb).wait_recv()
            s2buf[qb] = s1buf[qb] + r1buf[qb]
            rdma2(qb).start()

        for qb in range(NQB):
            accq = jnp.zeros((BLK, DM), jnp.float32)
            for h in range(H_LOC):
                if qb == 0:
                    k_copies[h].wait()
                    v_copies[h].wait()
                q = q_all[qb * BLK:(qb + 1) * BLK, h * DH:(h + 1) * DH]
                s = lax.dot_general(q, k_buf[h], (((1,), (1,)), ((), ())),
                                    preferred_element_type=jnp.float32)
                e = jnp.exp(s + biases[qb])
                r = jnp.sum(e, axis=1, keepdims=True)
                ctx = jnp.dot(e, v_buf[h],
                              preferred_element_type=jnp.float32) / r
                accq = accq + jnp.dot(ctx, wo_ref[h * DH:(h + 1) * DH, :],
                                      preferred_element_type=jnp.float32)
            s1buf[qb] = accq
            rdma1(qb).start()
            if qb >= 1:
                stage2_issue(qb - 1)
        stage2_issue(NQB - 1)

        for qb in range(NQB):
            rdma2(qb).wait()
            out_ref[0, qb * BLK:(qb + 1) * BLK, :] = s2buf[qb] + r2buf[qb]
        for qb in range(NQB):
            rdma1(qb).wait_send()

    return pl.pallas_call(
        body,
        out_shape=jax.ShapeDtypeStruct((1, SQ, DM), jnp.float32),
        in_specs=[
            pl.BlockSpec(memory_space=pltpu.VMEM),
            pl.BlockSpec(memory_space=pltpu.VMEM),
            pl.BlockSpec(memory_space=pl.ANY),
            pl.BlockSpec(memory_space=pl.ANY),
            pl.BlockSpec(memory_space=pltpu.VMEM),
        ],
        out_specs=pl.BlockSpec(memory_space=pltpu.VMEM),
        scratch_shapes=[
            pltpu.VMEM((H_LOC, SKV, DH), jnp.float32),
            pltpu.VMEM((H_LOC, SKV, DH), jnp.float32),
            pltpu.SemaphoreType.DMA((H_LOC,)),
            pltpu.SemaphoreType.DMA((H_LOC,)),
            pltpu.VMEM((NQB, BLK, DM), jnp.float32),
            pltpu.VMEM((NQB, BLK, DM), jnp.float32),
            pltpu.VMEM((NQB, BLK, DM), jnp.float32),
            pltpu.VMEM((NQB, BLK, DM), jnp.float32),
            pltpu.SemaphoreType.DMA((NQB,)),
            pltpu.SemaphoreType.DMA((NQB,)),
            pltpu.SemaphoreType.DMA((NQB,)),
            pltpu.SemaphoreType.DMA((NQB,)),
        ],
        compiler_params=pltpu.CompilerParams(
            collective_id=0, vmem_limit_bytes=60 * 1024 * 1024),
    )(x, Wq, K_ext, V_ext, Wo)
